# baseline (speedup 1.0000x reference)
import functools

import jax
import jax.numpy as jnp
from jax import lax
from jax.experimental import pallas as pl
from jax.experimental.pallas import tpu as pltpu
from jax.experimental.pallas import tpu_sc as plsc

_B, _M, _S, _E, _V = 1024, 50, 20, 32, 100000
_NW = 32
_CR = 500
_NC = (_B * _M * _S) // (_NW * _CR)
_TOTC = _B * _M * _S // _CR
_JG = 4
_GSZ = _CR // _JG
_NBUF = 4
_MT = 5
_VSL = _V // 16


def _position_encoding(sent_size, emb_size):
    j = jnp.arange(1, sent_size + 1, dtype=jnp.float32)[:, None]
    k = jnp.arange(1, emb_size + 1, dtype=jnp.float32)[None, :]
    return (1.0 - j / sent_size) - (k / emb_size) * (1.0 - 2.0 * j / sent_size)


def _sc_body(x_hbm, pe_hbm, tf_hbm, w_hbm, out_hbm,
             idx_v, bbuf_v, fbuf_v, pe_v, tf_v,
             isems, gsems, ssems):
    sid = lax.axis_index("s")
    wid = sid * 2 + lax.axis_index("c")
    c_base = wid * _NC

    pltpu.sync_copy(pe_hbm, pe_v)
    pltpu.sync_copy(tf_hbm, tf_v)

    def issue_idx(c, b):
        pltpu.async_copy(x_hbm.at[c_base + c], idx_v.at[b], isems[b])

    def wait_idx(b):
        pltpu.make_async_copy(x_hbm.at[0], idx_v.at[b], isems[b]).wait()

    def issue_gathers(c, b):
        for j in range(_JG):
            pltpu.async_copy(w_hbm.at[idx_v.at[b, j]],
                             bbuf_v.at[b, pl.ds(j * _GSZ, _GSZ)], gsems[b])

    def wait_gathers(b):
        pltpu.make_async_copy(w_hbm.at[pl.ds(0, _CR)], bbuf_v.at[b],
                              gsems[b]).wait()

    def wait_scatter(c, b):
        pltpu.make_async_copy(fbuf_v.at[b], out_hbm.at[c_base + c],
                              ssems[b]).wait()

    def compute(c, b):
        m_base = (c % 2) * (_CR // _S)

        def mt_body(mt, _):
            m0 = mt * _MT
            tf_regs = []
            for k in range(_MT):
                tf_regs.append((tf_v[m_base + m0 + k, pl.ds(0, 16)],
                                tf_v[m_base + m0 + k, pl.ds(16, 16)]))

            def s_body(s, _):
                pe0 = pe_v[s, pl.ds(0, 16)]
                pe1 = pe_v[s, pl.ds(16, 16)]
                for k in range(_MT):
                    r = (m0 + k) * _S + s
                    t0, t1 = tf_regs[k]
                    v0, v1 = plsc.unpack(bbuf_v[b, r, :],
                                         format=plsc.PackFormat.INTERLEAVED,
                                         preferred_element_type=jnp.float32)
                    fbuf_v[b, r, pl.ds(0, 16)] = v0 * pe0 + t0
                    fbuf_v[b, r, pl.ds(16, 16)] = v1 * pe1 + t1
                return 0

            lax.fori_loop(0, _S, s_body, 0)
            return 0

        lax.fori_loop(0, (_CR // _S) // _MT, mt_body, 0)

    for c0 in range(_NBUF):
        issue_idx(c0, c0)
    wait_idx(0)
    issue_gathers(0, 0)

    def phase(c, b):
        b1 = (b + 1) % _NBUF

        wait_gathers(b)

        @pl.when(c + _NBUF < _NC)
        def _():
            issue_idx(c + _NBUF, b)

        @pl.when(c + 1 < _NC)
        def _():
            wait_idx(b1)
            issue_gathers(c + 1, b1)

        @pl.when(c >= _NBUF)
        def _():
            wait_scatter(c - _NBUF, b)

        compute(c, b)
        pltpu.async_copy(fbuf_v.at[b], out_hbm.at[c_base + c], ssems[b])

    def chunk_body(t, _):
        for jb in range(_NBUF):
            phase(t * _NBUF + jb, jb)
        return 0

    lax.fori_loop(0, _NC // _NBUF, chunk_body, 0)

    for c in range(_NC - _NBUF, _NC):
        wait_scatter(c, c % _NBUF)


@jax.jit
def kernel(x, W_emb, W_temp):
    pe = _position_encoding(_S, _E)
    x3 = x.reshape(_TOTC, _JG, _GSZ).astype(jnp.int32)
    perm = jnp.stack([jnp.arange(16), jnp.arange(16) + 16], axis=1).reshape(-1)
    w_bf = W_emb[:, perm].astype(jnp.bfloat16)

    mesh = plsc.VectorSubcoreMesh(core_axis_name="c", subcore_axis_name="s")
    run = pl.kernel(
        _sc_body,
        out_type=jax.ShapeDtypeStruct((_TOTC, _CR, _E), jnp.float32),
        mesh=mesh,
        scratch_types=[
            pltpu.VMEM((_NBUF, _JG, _GSZ), jnp.int32),
            pltpu.VMEM((_NBUF, _CR, _E), jnp.bfloat16),
            pltpu.VMEM((_NBUF, _CR, _E), jnp.float32),
            pltpu.VMEM((_S, _E), jnp.float32),
            pltpu.VMEM((_M, _E), jnp.float32),
            [pltpu.SemaphoreType.DMA] * _NBUF,
            [pltpu.SemaphoreType.DMA] * _NBUF,
            [pltpu.SemaphoreType.DMA] * _NBUF,
        ],
        compiler_params=pltpu.CompilerParams(use_tc_tiling_on_sc=False,
                                             needs_layout_passes=False),
    )
    out = run(x3, pe, W_temp, w_bf)
    return out.reshape(_B, _M, _S, _E)


# speedup vs baseline: 1.4302x; 1.4302x over previous
import functools

import jax
import jax.numpy as jnp
from jax import lax
from jax.experimental import pallas as pl
from jax.experimental.pallas import tpu as pltpu
from jax.experimental.pallas import tpu_sc as plsc

_B, _M, _S, _E, _V = 1024, 50, 20, 32, 100000
_NW = 32
_CR = 500
_NC = (_B * _M * _S) // (_NW * _CR)
_TOTC = _B * _M * _S // _CR
_JG = 4
_GSZ = _CR // _JG
_NBUF = 4
_BB = 16


def _position_encoding(sent_size, emb_size):
    j = jnp.arange(1, sent_size + 1, dtype=jnp.float32)[:, None]
    k = jnp.arange(1, emb_size + 1, dtype=jnp.float32)[None, :]
    return (1.0 - j / sent_size) - (k / emb_size) * (1.0 - 2.0 * j / sent_size)


def _sc_body(x_hbm, w_hbm, g_hbm, idx_v, bbuf_v, isems, gsems, ssems):
    wid = lax.axis_index("s") * 2 + lax.axis_index("c")
    c_base = wid * _NC

    def issue_idx(c, b):
        pltpu.async_copy(x_hbm.at[c_base + c], idx_v.at[b], isems[b])

    def wait_idx(b):
        pltpu.make_async_copy(x_hbm.at[0], idx_v.at[b], isems[b]).wait()

    def issue_gathers(c, b):
        for j in range(_JG):
            pltpu.async_copy(w_hbm.at[idx_v.at[b, j]],
                             bbuf_v.at[b, pl.ds(j * _GSZ, _GSZ)], gsems[b])

    def wait_gathers(b):
        pltpu.make_async_copy(w_hbm.at[pl.ds(0, _CR)], bbuf_v.at[b],
                              gsems[b]).wait()

    def wait_scatter(c, b):
        pltpu.make_async_copy(bbuf_v.at[b], g_hbm.at[c_base + c],
                              ssems[b]).wait()

    for c0 in range(_NBUF):
        issue_idx(c0, c0)
    wait_idx(0)
    issue_gathers(0, 0)

    def phase(c, b):
        b1 = (b + 1) % _NBUF

        wait_gathers(b)

        @pl.when(c + _NBUF < _NC)
        def _():
            issue_idx(c + _NBUF, b)

        pltpu.async_copy(bbuf_v.at[b], g_hbm.at[c_base + c], ssems[b])

        @pl.when(c + 1 < _NC)
        def _():
            wait_idx(b1)

            @pl.when(c + 1 >= _NBUF)
            def _():
                wait_scatter(c + 1 - _NBUF, b1)

            issue_gathers(c + 1, b1)

    def chunk_body(t, _):
        for jb in range(_NBUF):
            phase(t * _NBUF + jb, jb)
        return 0

    lax.fori_loop(0, _NC // _NBUF, chunk_body, 0)

    for c in range(_NC - _NBUF, _NC):
        wait_scatter(c, c % _NBUF)


def _tc_body(g_ref, pef_ref, tff_ref, out_ref):
    out_ref[...] = (g_ref[...].astype(jnp.float32) * pef_ref[None]
                    + tff_ref[None])


@jax.jit
def kernel(x, W_emb, W_temp):
    pe = _position_encoding(_S, _E)
    x3 = x.reshape(_TOTC, _JG, _GSZ).astype(jnp.int32)
    w_bf = W_emb.astype(jnp.bfloat16)

    mesh = plsc.VectorSubcoreMesh(core_axis_name="c", subcore_axis_name="s")
    gather_rows = pl.kernel(
        _sc_body,
        out_type=jax.ShapeDtypeStruct((_TOTC, _CR, _E), jnp.bfloat16),
        mesh=mesh,
        scratch_types=[
            pltpu.VMEM((_NBUF, _JG, _GSZ), jnp.int32),
            pltpu.VMEM((_NBUF, _CR, _E), jnp.bfloat16),
            [pltpu.SemaphoreType.DMA] * _NBUF,
            [pltpu.SemaphoreType.DMA] * _NBUF,
            [pltpu.SemaphoreType.DMA] * _NBUF,
        ],
        compiler_params=pltpu.CompilerParams(use_tc_tiling_on_sc=False),
    )
    g = gather_rows(x3, w_bf).reshape(_B, _M * _S * _E // 128, 128)

    pef = jnp.broadcast_to(pe[None, :, :], (_M, _S, _E)).reshape(-1, 128)
    tff = jnp.broadcast_to(W_temp[:, None, :], (_M, _S, _E)).reshape(-1, 128)
    _R128 = _M * _S * _E // 128

    out = pl.pallas_call(
        _tc_body,
        out_shape=jax.ShapeDtypeStruct((_B, _R128, 128), jnp.float32),
        grid=(_B // _BB,),
        in_specs=[
            pl.BlockSpec((_BB, _R128, 128), lambda i: (i, 0, 0)),
            pl.BlockSpec((_R128, 128), lambda i: (0, 0)),
            pl.BlockSpec((_R128, 128), lambda i: (0, 0)),
        ],
        out_specs=pl.BlockSpec((_BB, _R128, 128), lambda i: (i, 0, 0)),
    )(g, pef, tff)
    return out.reshape(_B, _M, _S, _E)
